# baseline (speedup 1.0000x reference)
import functools

import jax
import jax.numpy as jnp
from jax import lax
from jax.experimental import pallas as pl
from jax.experimental.pallas import tpu as pltpu
from jax.experimental.pallas import tpu_sc as plsc

_N = 10000
_D = 128
_E = 320000

_NC = 2
_NS = 16
_LANES = 16

_HALF = _N // _NC
_NDUM = 8
_EPT = (2 * _E) // _NS
_CH = 320
_NCH = _EPT // _CH
_ZROWS = (_HALF + _NDUM) // _NS


def _mm_body(x_ref, w_ref, y_ref):
    xb = x_ref[...]
    y_ref[0] = jnp.dot(xb, w_ref[0], preferred_element_type=jnp.float32)
    y_ref[1] = jnp.dot(xb, w_ref[1], preferred_element_type=jnp.float32)


def _project(x, w2):
    blk = 1000
    return pl.pallas_call(
        _mm_body,
        grid=(_N // blk,),
        in_specs=[
            pl.BlockSpec((blk, _D), lambda i: (i, 0)),
            pl.BlockSpec((2, _D, _D), lambda i: (0, 0, 0)),
        ],
        out_specs=pl.BlockSpec((2, blk, _D), lambda i: (0, i, 0)),
        out_shape=jax.ShapeDtypeStruct((2, _N, _D), jnp.float32),
    )(x, w2)


def _spmm_sc(ycat, dstcat, srccat, ewcat):
    mesh = plsc.VectorSubcoreMesh(core_axis_name="c", subcore_axis_name="s")

    @functools.partial(
        pl.kernel,
        out_type=jax.ShapeDtypeStruct((_NC, _HALF, _D), jnp.float32),
        mesh=mesh,
        scratch_types=[
            [pltpu.VMEM((_CH,), jnp.int32)] * 2,
            [pltpu.VMEM((_CH,), jnp.int32)] * 2,
            [pltpu.VMEM((_CH,), jnp.float32)] * 2,
            [pltpu.VMEM((_CH,), jnp.int32)] * 2,
            pltpu.VMEM((2, _CH, _D), jnp.float32),
            pltpu.VMEM_SHARED((_HALF + _NDUM, _D), jnp.float32),
            [pltpu.SemaphoreType.DMA] * 2,
            [pltpu.SemaphoreType.DMA] * 2,
        ],
    )
    def spmm(y_hbm, dst_hbm, src_hbm, ew_hbm, out_hbm, dst_v, src_v, w_v,
             sdst_v, rows_v, acc_sh, se, sg):
        c = lax.axis_index("c")
        s = lax.axis_index("s")
        lo = c * _HALF
        lane = lax.iota(jnp.int32, _LANES)

        @pl.loop(0, _ZROWS)
        def _zero_rows(e):
            for j in range(_D // _LANES):
                rows_v[0, e, pl.ds(j * _LANES, _LANES)] = jnp.zeros(
                    (_LANES,), jnp.float32)

        pltpu.sync_copy(
            rows_v.at[0, pl.ds(0, _ZROWS)],
            acc_sh.at[pl.ds(s * _ZROWS, _ZROWS)])

        plsc.subcore_barrier()

        rel_off = jnp.where(s >= _NS // 2, _N, 0)
        ebase = s * _EPT

        @pl.loop(0, _NCH)
        def _chunk(k):
            base = ebase + k * _CH
            pltpu.sync_copy(dst_hbm.at[pl.ds(base, _CH)], dst_v[0])
            pltpu.sync_copy(src_hbm.at[pl.ds(base, _CH)], src_v[0])
            pltpu.sync_copy(ew_hbm.at[pl.ds(base, _CH)], w_v[0])

            @pl.loop(0, _CH // _LANES)
            def _fix(i):
                sl = pl.ds(i * _LANES, _LANES)
                src_v[0][sl] = src_v[0][sl] + rel_off
                d = dst_v[0][sl] - lo
                mine = (d >= 0) & (d < _HALF)
                sdst_v[0][sl] = jnp.where(
                    mine, d, _HALF + (lane & (_NDUM - 1)))

            pltpu.async_copy(y_hbm.at[src_v[0]], rows_v.at[0], sg[0]).wait()

            @pl.loop(0, _CH // _LANES)
            def _scale(g):
                w16 = w_v[0][pl.ds(g * _LANES, _LANES)]
                for l in range(_LANES):
                    e = g * _LANES + l
                    wv = jnp.full((_LANES,), w16[l], jnp.float32)
                    for j in range(_D // _LANES):
                        sl = pl.ds(j * _LANES, _LANES)
                        rows_v[0, e, sl] = rows_v[0, e, sl] * wv

            pltpu.sync_copy(rows_v.at[0], acc_sh.at[sdst_v[0]], add=True)

        plsc.subcore_barrier()

        eblk = 312

        @pl.when(s < _NS - 1)
        def _copy_main():
            pltpu.sync_copy(
                acc_sh.at[pl.ds(s * eblk, eblk)],
                out_hbm.at[c, pl.ds(s * eblk, eblk)])

        @pl.when(s == _NS - 1)
        def _copy_tail():
            tail = _HALF - (_NS - 1) * eblk
            pltpu.sync_copy(
                acc_sh.at[pl.ds((_NS - 1) * eblk, tail)],
                out_hbm.at[c, pl.ds((_NS - 1) * eblk, tail)])

    return spmm(ycat, dstcat, srccat, ewcat)


def _combine_body(p_ref, o_ref):
    o_ref[...] = jnp.maximum(p_ref[0], 0.0)


def _combine(partials):
    blk = 1000
    return pl.pallas_call(
        _combine_body,
        grid=(_N // blk,),
        in_specs=[pl.BlockSpec(
            (1, blk, _D), lambda i: (i // (_HALF // 1000), i % (_HALF // 1000), 0))],
        out_specs=pl.BlockSpec((blk, _D), lambda i: (i, 0)),
        out_shape=jax.ShapeDtypeStruct((_N, _D), jnp.float32),
    )(partials)


def kernel(x, edge_index_r0, edge_weight_r0, edge_index_r1, edge_weight_r1, W):
    y = _project(x, W.reshape(2, _D, _D))
    ycat = y.reshape(2 * _N, _D)
    dstcat = jnp.concatenate([edge_index_r0[0], edge_index_r1[0]])
    srccat = jnp.concatenate([edge_index_r0[1], edge_index_r1[1]])
    ewcat = jnp.concatenate([edge_weight_r0, edge_weight_r1])
    partials = _spmm_sc(ycat, dstcat, srccat, ewcat)
    return _combine(partials)


# speedup vs baseline: 1.0563x; 1.0563x over previous
import functools

import jax
import jax.numpy as jnp
from jax import lax
from jax.experimental import pallas as pl
from jax.experimental.pallas import tpu as pltpu
from jax.experimental.pallas import tpu_sc as plsc

_N = 10000
_D = 128
_E = 320000

_NC = 2
_NS = 16
_LANES = 16

_HALF = _N // _NC
_NDUM = 8
_EPT = (2 * _E) // _NS
_CH = 400
_NCH = _EPT // _CH
_ZROWS = (_HALF + _NDUM) // _NS


def _mm_body(x_ref, w_ref, y_ref):
    xb = x_ref[...]
    y_ref[0] = jnp.dot(xb, w_ref[0], preferred_element_type=jnp.float32)
    y_ref[1] = jnp.dot(xb, w_ref[1], preferred_element_type=jnp.float32)


def _project(x, w2):
    blk = 1000
    return pl.pallas_call(
        _mm_body,
        grid=(_N // blk,),
        in_specs=[
            pl.BlockSpec((blk, _D), lambda i: (i, 0)),
            pl.BlockSpec((2, _D, _D), lambda i: (0, 0, 0)),
        ],
        out_specs=pl.BlockSpec((2, blk, _D), lambda i: (0, i, 0)),
        out_shape=jax.ShapeDtypeStruct((2, _N, _D), jnp.float32),
    )(x, w2)


def _spmm_sc(ycat, dstcat, srccat, ewcat):
    mesh = plsc.VectorSubcoreMesh(core_axis_name="c", subcore_axis_name="s")

    @functools.partial(
        pl.kernel,
        out_type=jax.ShapeDtypeStruct((_NC, _HALF, _D), jnp.float32),
        mesh=mesh,
        scratch_types=[
            [pltpu.VMEM((_CH,), jnp.int32)] * 1,
            [pltpu.VMEM((_CH,), jnp.int32)] * 1,
            [pltpu.VMEM((_CH,), jnp.float32)] * 1,
            [pltpu.VMEM((_CH,), jnp.int32)] * 1,
            pltpu.VMEM((1, _CH, _D), jnp.float32),
            pltpu.VMEM_SHARED((_HALF + _NDUM, _D), jnp.float32),
            [pltpu.SemaphoreType.DMA] * 1,
            [pltpu.SemaphoreType.DMA] * 1,
        ],
    )
    def spmm(y_hbm, dst_hbm, src_hbm, ew_hbm, out_hbm, dst_v, src_v, w_v,
             sdst_v, rows_v, acc_sh, se, sg):
        c = lax.axis_index("c")
        s = lax.axis_index("s")
        lo = c * _HALF
        lane = lax.iota(jnp.int32, _LANES)

        @pl.loop(0, _ZROWS)
        def _zero_rows(e):
            for j in range(_D // _LANES):
                rows_v[0, e, pl.ds(j * _LANES, _LANES)] = jnp.zeros(
                    (_LANES,), jnp.float32)

        pltpu.sync_copy(
            rows_v.at[0, pl.ds(0, _ZROWS)],
            acc_sh.at[pl.ds(s * _ZROWS, _ZROWS)])

        plsc.subcore_barrier()

        rel_off = jnp.where(s >= _NS // 2, _N, 0)
        ebase = s * _EPT

        @pl.loop(0, _NCH)
        def _chunk(k):
            base = ebase + k * _CH
            pltpu.sync_copy(dst_hbm.at[pl.ds(base, _CH)], dst_v[0])
            pltpu.sync_copy(src_hbm.at[pl.ds(base, _CH)], src_v[0])
            pltpu.sync_copy(ew_hbm.at[pl.ds(base, _CH)], w_v[0])

            @pl.loop(0, _CH // _LANES)
            def _fix(i):
                sl = pl.ds(i * _LANES, _LANES)
                src_v[0][sl] = src_v[0][sl] + rel_off
                d = dst_v[0][sl] - lo
                mine = (d >= 0) & (d < _HALF)
                sdst_v[0][sl] = jnp.where(
                    mine, d, _HALF + (lane & (_NDUM - 1)))

            pltpu.async_copy(y_hbm.at[src_v[0]], rows_v.at[0], sg[0]).wait()

            @pl.loop(0, _CH // _LANES)
            def _scale(g):
                w16 = w_v[0][pl.ds(g * _LANES, _LANES)]
                for l in range(_LANES):
                    e = g * _LANES + l
                    wv = jnp.full((_LANES,), w16[l], jnp.float32)
                    for j in range(_D // _LANES):
                        sl = pl.ds(j * _LANES, _LANES)
                        rows_v[0, e, sl] = rows_v[0, e, sl] * wv

            pltpu.sync_copy(rows_v.at[0], acc_sh.at[sdst_v[0]], add=True)

        plsc.subcore_barrier()

        eblk = 312

        @pl.when(s < _NS - 1)
        def _copy_main():
            pltpu.sync_copy(
                acc_sh.at[pl.ds(s * eblk, eblk)],
                out_hbm.at[c, pl.ds(s * eblk, eblk)])

        @pl.when(s == _NS - 1)
        def _copy_tail():
            tail = _HALF - (_NS - 1) * eblk
            pltpu.sync_copy(
                acc_sh.at[pl.ds((_NS - 1) * eblk, tail)],
                out_hbm.at[c, pl.ds((_NS - 1) * eblk, tail)])

    return spmm(ycat, dstcat, srccat, ewcat)


def _combine_body(p_ref, o_ref):
    o_ref[...] = jnp.maximum(p_ref[0], 0.0)


def _combine(partials):
    blk = 1000
    return pl.pallas_call(
        _combine_body,
        grid=(_N // blk,),
        in_specs=[pl.BlockSpec(
            (1, blk, _D), lambda i: (i // (_HALF // 1000), i % (_HALF // 1000), 0))],
        out_specs=pl.BlockSpec((blk, _D), lambda i: (i, 0)),
        out_shape=jax.ShapeDtypeStruct((_N, _D), jnp.float32),
    )(partials)


def kernel(x, edge_index_r0, edge_weight_r0, edge_index_r1, edge_weight_r1, W):
    y = _project(x, W.reshape(2, _D, _D))
    ycat = y.reshape(2 * _N, _D)
    dstcat = jnp.concatenate([edge_index_r0[0], edge_index_r1[0]])
    srccat = jnp.concatenate([edge_index_r0[1], edge_index_r1[1]])
    ewcat = jnp.concatenate([edge_weight_r0, edge_weight_r1])
    partials = _spmm_sc(ycat, dstcat, srccat, ewcat)
    return _combine(partials)
